# Initial kernel scaffold; baseline (speedup 1.0000x reference)
import jax
import jax.numpy as jnp
from jax.experimental import pallas as pl


def kernel(x, W, expert_bases, log_lambda, sips_alpha, sips_beta):
    raise NotImplementedError("write your pallas kernel here")


# speedup vs baseline: 3.2410x; 3.2410x over previous
import dataclasses
import functools

import jax
import jax.numpy as jnp
from jax import lax
from jax.experimental import pallas as pl
from jax.experimental.pallas import tpu as pltpu
from jax.experimental.pallas import tpu_sc as plsc

_T_BLK = 512
_NW = 32
_LANES = 16


def _score_body(ll_ref, a_ref, b_ref, x_ref, wt_ref, bt_ref, g_ref, out_ref):
    z = jnp.dot(x_ref[...], wt_ref[...], preferred_element_type=jnp.float32,
                precision=lax.Precision.DEFAULT)
    proj = jnp.dot(z, bt_ref[...], preferred_element_type=jnp.float32,
                   precision=lax.Precision.DEFAULT)
    psq = proj * proj
    aff = jnp.dot(psq, g_ref[...], preferred_element_type=jnp.float32,
                  precision=lax.Precision.HIGHEST)
    lam = jnp.log(1.0 + jnp.exp(ll_ref[...]))
    aff = aff * lam
    scores = a_ref[0] * jnp.tanh(aff / b_ref[0])
    out_ref[...] = scores.T


def _make_aux_body(n_tok, n_experts):
    uniform = 1.0 / n_experts

    def body(p_ref, o_ref):
        frac = jnp.sum(p_ref[...], axis=0, keepdims=True) / n_tok
        d = frac - uniform
        o_ref[...] = jnp.sum(d * d, axis=1, keepdims=True)

    return body


def _route(scores_t, n_tok, n_experts, top_k):
    chunk = n_tok // _NW
    n_groups = chunk // _LANES
    mesh = plsc.VectorSubcoreMesh(core_axis_name="c", subcore_axis_name="s")
    cp = pltpu.CompilerParams()
    if "needs_layout_passes" in pltpu.CompilerParams.__dataclass_fields__:
        cp = dataclasses.replace(cp, needs_layout_passes=False)

    @functools.partial(
        pl.kernel,
        compiler_params=cp,
        out_type=(
            jax.ShapeDtypeStruct((n_tok, n_experts), jnp.float32),
            jax.ShapeDtypeStruct((n_tok, top_k), jnp.int32),
            jax.ShapeDtypeStruct((_NW, n_experts), jnp.float32),
        ),
        mesh=mesh,
        scratch_types=[
            pltpu.VMEM((n_experts, chunk), jnp.float32),
            pltpu.VMEM((chunk, n_experts), jnp.float32),
            pltpu.VMEM((chunk, top_k), jnp.int32),
            pltpu.VMEM((n_experts,), jnp.float32),
        ],
    )
    def route_kernel(s_hbm, w_hbm, i_hbm, p_hbm, sbuf, wbuf, ibuf, esum):
        wid = lax.axis_index("s") * 2 + lax.axis_index("c")
        base = wid * chunk
        pltpu.sync_copy(s_hbm.at[:, pl.ds(base, chunk)], sbuf)

        zero16 = jnp.zeros((_LANES,), jnp.float32)

        @pl.loop(0, chunk)
        def _zero(r):
            for c4 in range(n_experts // _LANES):
                wbuf[r, pl.ds(c4 * _LANES, _LANES)] = zero16

        @pl.loop(0, n_groups)
        def _group(g):
            col = g * _LANES
            rows = col + lax.iota(jnp.int32, _LANES)
            neg = jnp.full((_LANES,), -jnp.inf, jnp.float32)
            izero = jnp.zeros((_LANES,), jnp.int32)
            init = tuple([neg] * top_k + [izero] * top_k)

            def body(e, carry):
                vs = list(carry[:top_k])
                ii = list(carry[top_k:])
                cur_v = sbuf[e, pl.ds(col, _LANES)]
                cur_i = jnp.full((_LANES,), e, jnp.int32)
                ins = jnp.zeros((_LANES,), jnp.bool_)
                for j in range(top_k):
                    take = jnp.logical_or(ins, cur_v > vs[j])
                    nv = jnp.where(take, cur_v, vs[j])
                    ni = jnp.where(take, cur_i, ii[j])
                    cur_v = jnp.where(take, vs[j], cur_v)
                    cur_i = jnp.where(take, ii[j], cur_i)
                    vs[j] = nv
                    ii[j] = ni
                    ins = take
                return tuple(vs + ii)

            fin = lax.fori_loop(0, n_experts, body, init)
            vs = fin[:top_k]
            ii = fin[top_k:]
            m = vs[0]
            es = [jnp.exp(v - m) for v in vs]
            tot = es[0]
            for j in range(1, top_k):
                tot = tot + es[j]
            inv = 1.0 / tot
            for j in range(top_k):
                plsc.store_scatter(wbuf, [rows, ii[j]], es[j] * inv)
                plsc.store_scatter(
                    ibuf, [rows, jnp.full((_LANES,), j, jnp.int32)], ii[j])

        def csum_body(r, acc):
            return tuple(acc[c4] + wbuf[r, pl.ds(c4 * _LANES, _LANES)]
                         for c4 in range(n_experts // _LANES))

        accs = lax.fori_loop(0, chunk, csum_body,
                             tuple(zero16 for _ in range(n_experts // _LANES)))
        for c4 in range(n_experts // _LANES):
            esum[pl.ds(c4 * _LANES, _LANES)] = accs[c4]

        pltpu.sync_copy(wbuf, w_hbm.at[pl.ds(base, chunk)])
        pltpu.sync_copy(ibuf, i_hbm.at[pl.ds(base, chunk)])
        pltpu.sync_copy(esum, p_hbm.at[wid])

    return route_kernel(scores_t)


def kernel(x, W, expert_bases, log_lambda, sips_alpha, sips_beta):
    B, T, D = x.shape
    E, L, K = expert_bases.shape
    top_k = 8
    n_tok = B * T

    xf = x.reshape(n_tok, D)
    wt = W.T
    bt = expert_bases.transpose(1, 0, 2).reshape(L, E * K)
    gmat = jnp.repeat(jnp.eye(E, dtype=jnp.float32), K, axis=0)
    ll2 = log_lambda.reshape(1, E)

    scores_t = pl.pallas_call(
        _score_body,
        grid=(n_tok // _T_BLK,),
        in_specs=[
            pl.BlockSpec((1, E), lambda i: (0, 0)),
            pl.BlockSpec(memory_space=pltpu.SMEM),
            pl.BlockSpec(memory_space=pltpu.SMEM),
            pl.BlockSpec((_T_BLK, D), lambda i: (i, 0)),
            pl.BlockSpec((D, L), lambda i: (0, 0)),
            pl.BlockSpec((L, E * K), lambda i: (0, 0)),
            pl.BlockSpec((E * K, E), lambda i: (0, 0)),
        ],
        out_specs=pl.BlockSpec((E, _T_BLK), lambda i: (0, i)),
        out_shape=jax.ShapeDtypeStruct((E, n_tok), jnp.float32),
    )(ll2, sips_alpha, sips_beta, xf, wt, bt, gmat)

    wflat, iflat, parts = _route(scores_t, n_tok, E, top_k)

    aux_arr = pl.pallas_call(
        _make_aux_body(n_tok, E),
        in_specs=[pl.BlockSpec((_NW, E), lambda: (0, 0))],
        out_specs=pl.BlockSpec((1, 1), lambda: (0, 0)),
        out_shape=jax.ShapeDtypeStruct((1, 1), jnp.float32),
    )(parts)

    return (wflat.reshape(B, T, E),
            iflat.reshape(B, T, top_k),
            aux_arr.reshape(()))
